# baseline (speedup 1.0000x reference)
import functools

import jax
import jax.numpy as jnp
from jax import lax
from jax.experimental import pallas as pl
from jax.experimental.pallas import tpu as pltpu
from jax.experimental.pallas import tpu_sc as plsc

_BATCH = 4096
_FIELDS = 26
_EMB_DIM = 32
_VOCAB = 1000002
_VPAD = 1000064
_PROWS = _VPAD // 4
_TCOLS = _VPAD // 128
_NC, _NS = 2, 16
_NW = _NC * _NS
_NB = _BATCH // 128
_TILES = _FIELDS * _NB
_TPW = _TILES // _NW

_TAIL = _VOCAB - 128
_TAIL_ROW = (_TCOLS - 1) * 32

_mesh = plsc.VectorSubcoreMesh(core_axis_name="c", subcore_axis_name="s")
_params = pltpu.CompilerParams(needs_layout_passes=False)


_CC = 6
_NCHUNK = (_TCOLS + _CC - 1) // _CC
_CPW = (_NCHUNK + _NW - 1) // _NW


@functools.partial(
    pl.kernel,
    out_type=jax.ShapeDtypeStruct((_NCHUNK * 32 * _CC, 128), jnp.float32),
    mesh=_mesh,
    scratch_types=[
        pltpu.VMEM((_EMB_DIM, 128 * _CC + 1), jnp.float32),
        pltpu.VMEM((_EMB_DIM, 128 * _CC + 1), jnp.float32),
        pltpu.VMEM((32 * _CC, 128), jnp.float32),
        pltpu.VMEM((32 * _CC, 128), jnp.float32),
        pltpu.SemaphoreType.DMA,
        pltpu.SemaphoreType.DMA,
        pltpu.SemaphoreType.DMA,
        pltpu.SemaphoreType.DMA,
    ],
    compiler_params=_params,
)
def _sc_repack(tt_hbm, tail_hbm, tpk_hbm, vinA, vinB, voutA, voutB,
               sinA, sinB, soutA, soutB):
    wid = lax.axis_index("s") * _NC + lax.axis_index("c")
    liota = lax.iota(jnp.int32, 16)
    vins = (vinA, vinB)
    vouts = (voutA, voutB)
    sins = (sinA, sinB)
    souts = (soutA, soutB)

    def chunk_of(i):
        return wid + i * _NW

    def start_in(i, p):
        m = chunk_of(i)

        @pl.when(m < _NCHUNK - 1)
        def _full():
            pltpu.async_copy(
                tt_hbm.at[:, pl.ds(m * 128 * _CC, 128 * _CC)],
                vins[p].at[:, pl.ds(0, 128 * _CC)],
                sins[p],
            )

        @pl.when(m == _NCHUNK - 1)
        def _tail():
            pltpu.async_copy(
                tail_hbm, vins[p].at[:, pl.ds(0, 128)], sins[p]
            )

    def wait_in(i, p):
        m = chunk_of(i)

        @pl.when(m < _NCHUNK - 1)
        def _full():
            pltpu.make_async_copy(
                tt_hbm.at[:, pl.ds(0, 128 * _CC)],
                vins[p].at[:, pl.ds(0, 128 * _CC)],
                sins[p],
            ).wait()

        @pl.when(m == _NCHUNK - 1)
        def _tail():
            pltpu.make_async_copy(
                tail_hbm, vins[p].at[:, pl.ds(0, 128)], sins[p]
            ).wait()

    def start_out(i, p):
        m = chunk_of(i)
        pltpu.async_copy(
            vouts[p], tpk_hbm.at[pl.ds(m * 32 * _CC, 32 * _CC), :],
            souts[p],
        )

    def wait_out(p):
        pltpu.make_async_copy(
            vouts[p], tpk_hbm.at[pl.ds(0, 32 * _CC), :], souts[p]
        ).wait()

    rv0 = liota
    rv1 = liota + 16

    def transpose(p):
        vin, vout = vins[p], vouts[p]
        cv0 = tuple(jnp.full((16,), t // 2, jnp.int32) for t in range(8))

        def rr(r, cvs):
            for c in range(_CC):
                loads = []
                for t in range(8):
                    rv = rv1 if (t % 2) else rv0
                    loads.append(
                        plsc.load_gather(vin, [rv, cvs[t] + 128 * c])
                    )
                for t in range(8):
                    vout[32 * c + r, pl.ds(16 * t, 16)] = loads[t]
            return tuple(cv + 4 for cv in cvs)

        lax.fori_loop(0, 32, rr, cv0)

    start_in(0, 0)
    start_in(1, 1)

    def outer(s, carry):
        for p in (0, 1):
            i = 2 * s + p
            m = chunk_of(i)
            valid = m < _NCHUNK

            @pl.when(valid)
            def _work():
                wait_in(i, p)
                transpose(p)

            @pl.when(valid & (s >= 1))
            def _drain():
                wait_out(p)

            @pl.when(valid)
            def _emit():
                start_out(i, p)

            @pl.when(chunk_of(i + 2) < _NCHUNK)
            def _next():
                start_in(i + 2, p)
        return carry

    lax.fori_loop(0, (_CPW + 1) // 2, outer, 0)
    wait_out(0)
    wait_out(1)


@functools.partial(
    pl.kernel,
    out_type=jax.ShapeDtypeStruct((_FIELDS, _EMB_DIM, _BATCH), jnp.float32),
    mesh=_mesh,
    scratch_types=[
        pltpu.VMEM((128,), jnp.int32),
        pltpu.VMEM((128,), jnp.int32),
        pltpu.VMEM((128,), jnp.int32),
        pltpu.VMEM((128, 129), jnp.float32),
        pltpu.VMEM((_EMB_DIM, 128), jnp.float32),
        pltpu.SemaphoreType.DMA,
    ],
    compiler_params=_params,
)
def _sc_gather(tpk_hbm, xt_hbm, out_hbm, idx_v, idx4_v, bcol_v, rows_v, out_v, sem):
    wid = lax.axis_index("s") * _NC + lax.axis_index("c")
    liota = lax.iota(jnp.int32, 16)

    def tile_body(t, carry):
        g = wid * _TPW + t
        f = g // _NB
        b = g % _NB
        pltpu.sync_copy(xt_hbm.at[f, pl.ds(b * 128, 128)], idx_v)

        def prep(q, c):
            v = idx_v[pl.ds(q * 16, 16)]
            is_tail = v >= _TAIL
            base = v - jnp.where(is_tail, _TAIL, 0)
            idx4_v[pl.ds(q * 16, 16)] = lax.shift_right_logical(
                base, 2
            ) + jnp.where(is_tail, _TAIL_ROW, 0)
            bcol_v[pl.ds(q * 16, 16)] = lax.shift_left(
                lax.bitwise_and(base, 3), 5
            )
            return c

        lax.fori_loop(0, 8, prep, 0)
        pltpu.async_copy(
            tpk_hbm.at[idx4_v], rows_v.at[:, pl.ds(0, 128)], sem
        ).wait()

        def dq(q, c):
            lanes = liota + q * 16
            cols = bcol_v[pl.ds(q * 16, 16)]
            for dg in range(_EMB_DIM // 8):
                loads = [
                    plsc.load_gather(rows_v, [lanes, cols + (8 * dg + k)])
                    for k in range(8)
                ]
                for k in range(8):
                    out_v[8 * dg + k, pl.ds(q * 16, 16)] = loads[k]
            return c

        lax.fori_loop(0, 8, dq, 0)
        pltpu.sync_copy(out_v, out_hbm.at[f, :, pl.ds(b * 128, 128)])
        return carry

    lax.fori_loop(0, _TPW, tile_body, 0)


def kernel(x, filters, table):
    tt = table.T
    xt = x.T
    tail = jax.lax.slice(table, (_TAIL, 0), (_VOCAB, _EMB_DIM)).T
    tpk = _sc_repack(tt, tail)
    out_t = _sc_gather(tpk, xt)
    return (out_t.transpose(2, 0, 1), filters)


# speedup vs baseline: 1.0638x; 1.0638x over previous
import functools

import jax
import jax.numpy as jnp
from jax import lax
from jax.experimental import pallas as pl
from jax.experimental.pallas import tpu as pltpu
from jax.experimental.pallas import tpu_sc as plsc

_BATCH = 4096
_FIELDS = 26
_EMB_DIM = 32
_VOCAB = 1000002
_VPAD = 1000064
_PROWS = _VPAD // 4
_TCOLS = _VPAD // 128
_NC, _NS = 2, 16
_NW = _NC * _NS
_NB = _BATCH // 128
_TILES = _FIELDS * _NB
_TPW = _TILES // _NW

_TAIL = _VOCAB - 128
_TAIL_ROW = (_TCOLS - 1) * 32

_mesh = plsc.VectorSubcoreMesh(core_axis_name="c", subcore_axis_name="s")
_params = pltpu.CompilerParams(needs_layout_passes=False)


_CC = 4
_NCHUNK = (_TCOLS + _CC - 1) // _CC
_CPW = (_NCHUNK + _NW - 1) // _NW


@functools.partial(
    pl.kernel,
    out_type=jax.ShapeDtypeStruct((_NCHUNK * 32 * _CC, 128), jnp.float32),
    mesh=_mesh,
    scratch_types=[
        pltpu.VMEM((_EMB_DIM, 128 * _CC + 8), jnp.float32),
        pltpu.VMEM((_EMB_DIM, 128 * _CC + 8), jnp.float32),
        pltpu.VMEM((32 * _CC, 128), jnp.float32),
        pltpu.VMEM((32 * _CC, 128), jnp.float32),
        pltpu.SemaphoreType.DMA,
        pltpu.SemaphoreType.DMA,
        pltpu.SemaphoreType.DMA,
        pltpu.SemaphoreType.DMA,
    ],
    compiler_params=_params,
)
def _sc_repack(tt_hbm, tail_hbm, tpk_hbm, vinA, vinB, voutA, voutB,
               sinA, sinB, soutA, soutB):
    wid = lax.axis_index("s") * _NC + lax.axis_index("c")
    liota = lax.iota(jnp.int32, 16)
    vins = (vinA, vinB)
    vouts = (voutA, voutB)
    sins = (sinA, sinB)
    souts = (soutA, soutB)

    def chunk_of(i):
        return wid + i * _NW

    def start_in(i, p):
        m = chunk_of(i)

        @pl.when(m < _NCHUNK - 1)
        def _full():
            pltpu.async_copy(
                tt_hbm.at[:, pl.ds(m * 128 * _CC, 128 * _CC)],
                vins[p].at[:, pl.ds(0, 128 * _CC)],
                sins[p],
            )

        @pl.when(m == _NCHUNK - 1)
        def _tail():
            pltpu.async_copy(
                tail_hbm, vins[p].at[:, pl.ds(0, 128)], sins[p]
            )

    def wait_in(i, p):
        m = chunk_of(i)

        @pl.when(m < _NCHUNK - 1)
        def _full():
            pltpu.make_async_copy(
                tt_hbm.at[:, pl.ds(0, 128 * _CC)],
                vins[p].at[:, pl.ds(0, 128 * _CC)],
                sins[p],
            ).wait()

        @pl.when(m == _NCHUNK - 1)
        def _tail():
            pltpu.make_async_copy(
                tail_hbm, vins[p].at[:, pl.ds(0, 128)], sins[p]
            ).wait()

    def start_out(i, p):
        m = chunk_of(i)
        pltpu.async_copy(
            vouts[p], tpk_hbm.at[pl.ds(m * 32 * _CC, 32 * _CC), :],
            souts[p],
        )

    def wait_out(p):
        pltpu.make_async_copy(
            vouts[p], tpk_hbm.at[pl.ds(0, 32 * _CC), :], souts[p]
        ).wait()

    rv0 = liota
    rv1 = liota + 16

    def transpose(p):
        vin, vout = vins[p], vouts[p]
        cv0 = tuple(jnp.full((16,), t // 2, jnp.int32) for t in range(8))

        def rr(r, cvs):
            for c in range(_CC):
                loads = []
                for t in range(8):
                    rv = rv1 if (t % 2) else rv0
                    loads.append(
                        plsc.load_gather(vin, [rv, cvs[t] + 128 * c])
                    )
                for t in range(8):
                    vout[32 * c + r, pl.ds(16 * t, 16)] = loads[t]
            return tuple(cv + 4 for cv in cvs)

        lax.fori_loop(0, 32, rr, cv0)

    start_in(0, 0)
    start_in(1, 1)

    def outer(s, carry):
        for p in (0, 1):
            i = 2 * s + p
            m = chunk_of(i)
            valid = m < _NCHUNK

            @pl.when(valid)
            def _work():
                wait_in(i, p)
                transpose(p)

            @pl.when(valid & (s >= 1))
            def _drain():
                wait_out(p)

            @pl.when(valid)
            def _emit():
                start_out(i, p)

            @pl.when(chunk_of(i + 2) < _NCHUNK)
            def _next():
                start_in(i + 2, p)
        return carry

    lax.fori_loop(0, (_CPW + 1) // 2, outer, 0)
    wait_out(0)
    wait_out(1)


@functools.partial(
    pl.kernel,
    out_type=jax.ShapeDtypeStruct((_FIELDS, _EMB_DIM, _BATCH), jnp.float32),
    mesh=_mesh,
    scratch_types=[
        pltpu.VMEM((128,), jnp.int32),
        pltpu.VMEM((128,), jnp.int32),
        pltpu.VMEM((128,), jnp.int32),
        pltpu.VMEM((128, 136), jnp.float32),
        pltpu.VMEM((_EMB_DIM, 128), jnp.float32),
        pltpu.SemaphoreType.DMA,
    ],
    compiler_params=_params,
)
def _sc_gather(tpk_hbm, xt_hbm, out_hbm, idx_v, idx4_v, bcol_v, rows_v, out_v, sem):
    wid = lax.axis_index("s") * _NC + lax.axis_index("c")
    liota = lax.iota(jnp.int32, 16)

    def tile_body(t, carry):
        g = wid * _TPW + t
        f = g // _NB
        b = g % _NB
        pltpu.sync_copy(xt_hbm.at[f, pl.ds(b * 128, 128)], idx_v)

        def prep(q, c):
            v = idx_v[pl.ds(q * 16, 16)]
            is_tail = v >= _TAIL
            base = v - jnp.where(is_tail, _TAIL, 0)
            idx4_v[pl.ds(q * 16, 16)] = lax.shift_right_logical(
                base, 2
            ) + jnp.where(is_tail, _TAIL_ROW, 0)
            bcol_v[pl.ds(q * 16, 16)] = lax.shift_left(
                lax.bitwise_and(base, 3), 5
            )
            return c

        lax.fori_loop(0, 8, prep, 0)
        pltpu.async_copy(
            tpk_hbm.at[idx4_v], rows_v.at[:, pl.ds(0, 128)], sem
        ).wait()

        def dq(q, c):
            lanes = liota + q * 16
            cols = bcol_v[pl.ds(q * 16, 16)]
            for dg in range(_EMB_DIM // 8):
                loads = [
                    plsc.load_gather(rows_v, [lanes, cols + (8 * dg + k)])
                    for k in range(8)
                ]
                for k in range(8):
                    out_v[8 * dg + k, pl.ds(q * 16, 16)] = loads[k]
            return c

        lax.fori_loop(0, 8, dq, 0)
        pltpu.sync_copy(out_v, out_hbm.at[f, :, pl.ds(b * 128, 128)])
        return carry

    lax.fori_loop(0, _TPW, tile_body, 0)


def kernel(x, filters, table):
    tt = table.T
    xt = x.T
    tail = jax.lax.slice(table, (_TAIL, 0), (_VOCAB, _EMB_DIM)).T
    tpk = _sc_repack(tt, tail)
    out_t = _sc_gather(tpk, xt)
    return (out_t.transpose(2, 0, 1), filters)


# speedup vs baseline: 1.0991x; 1.0331x over previous
import functools

import jax
import jax.numpy as jnp
from jax import lax
from jax.experimental import pallas as pl
from jax.experimental.pallas import tpu as pltpu
from jax.experimental.pallas import tpu_sc as plsc

_BATCH = 4096
_FIELDS = 26
_EMB_DIM = 32
_VOCAB = 1000002
_NC, _NS = 2, 16
_NW = _NC * _NS
_NB = _BATCH // 128
_TILES = _FIELDS * _NB
_TPW = _TILES // _NW

_mesh = plsc.VectorSubcoreMesh(core_axis_name="c", subcore_axis_name="s")
_params = pltpu.CompilerParams(
    needs_layout_passes=False, use_tc_tiling_on_sc=False
)


@functools.partial(
    pl.kernel,
    out_type=jax.ShapeDtypeStruct((_FIELDS, _EMB_DIM, _BATCH), jnp.float32),
    mesh=_mesh,
    scratch_types=[
        pltpu.VMEM((128,), jnp.int32),
        pltpu.VMEM((128, _EMB_DIM), jnp.float32),
        pltpu.VMEM((_EMB_DIM, 128), jnp.float32),
        pltpu.SemaphoreType.DMA,
    ],
    compiler_params=_params,
)
def _sc_gather(table_hbm, xt_hbm, out_hbm, idx_v, rows_v, out_v, sem):
    wid = lax.axis_index("s") * _NC + lax.axis_index("c")
    liota = lax.iota(jnp.int32, 16)

    def tile_body(t, carry):
        g = wid * _TPW + t
        f = g // _NB
        b = g % _NB
        pltpu.sync_copy(xt_hbm.at[f, pl.ds(b * 128, 128)], idx_v)
        pltpu.async_copy(table_hbm.at[idx_v], rows_v, sem).wait()

        def dq(q, c):
            lanes = liota + q * 16
            for dg in range(_EMB_DIM // 8):
                loads = [
                    plsc.load_gather(
                        rows_v,
                        [lanes, jnp.full((16,), 8 * dg + k, jnp.int32)],
                    )
                    for k in range(8)
                ]
                for k in range(8):
                    out_v[8 * dg + k, pl.ds(q * 16, 16)] = loads[k]
            return c

        lax.fori_loop(0, 8, dq, 0)
        pltpu.sync_copy(out_v, out_hbm.at[f, :, pl.ds(b * 128, 128)])
        return carry

    lax.fori_loop(0, _TPW, tile_body, 0)


def kernel(x, filters, table):
    xt = x.T
    out_t = _sc_gather(table, xt)
    return (out_t.transpose(2, 0, 1), filters)
